# Initial kernel scaffold; baseline (speedup 1.0000x reference)
import jax
import jax.numpy as jnp
from jax.experimental import pallas as pl


def kernel(x, se, seven_edge_index, edge_index, train_edge_id, gps_gcn_w, gps_gcn_b, attn_qkv_w, attn_qkv_b, attn_out_w, attn_out_b, mlp_w1, mlp_b1, mlp_w2, mlp_b2, gps_bn_g, gps_bn_b, lin_w, lin_b, gat_w, gat_b, gat_att_src, gat_att_dst, sub_w, sub_b, gcn2_w, gcn2_b, blk_bn_g, blk_bn_b, cls_w0, cls_b0, cls_w1, cls_b1, cls_w2, cls_b2, cls_w3, cls_b3, cls_w4, cls_b4, cls_w5, cls_b5):
    raise NotImplementedError("write your pallas kernel here")


# speedup vs baseline: 1.0010x; 1.0010x over previous
import jax
import jax.numpy as jnp
import numpy as np
from jax.experimental import pallas as pl
from jax.experimental.pallas import tpu as pltpu

N = 2048; E = 65536; C = 7; H = 128; SE = 20; D = H + SE; B = 4096; HEADS = 4


def _bn(x, g, b, eps=1e-5):
    m = jnp.mean(x, axis=0)
    v = jnp.var(x, axis=0)
    return g * (x - m) / jnp.sqrt(v + eps) + b


def _gcn(x, ei, W, b):
    n = x.shape[0]
    loops = jnp.arange(n, dtype=ei.dtype)
    src = jnp.concatenate([ei[0], loops])
    dst = jnp.concatenate([ei[1], loops])
    deg = jnp.zeros((n,), x.dtype).at[dst].add(1.0)
    dinv = 1.0 / jnp.sqrt(jnp.maximum(deg, 1e-12))
    norm = dinv[src] * dinv[dst]
    h = x @ W
    out = jnp.zeros_like(h).at[dst].add(h[src] * norm[:, None])
    return out + b


def _mha(x, Wqkv, bqkv, Wo, bo, heads):
    n, c = x.shape
    hd = c // heads
    q = (x @ Wqkv[0] + bqkv[0]).reshape(n, heads, hd).transpose(1, 0, 2)
    k = (x @ Wqkv[1] + bqkv[1]).reshape(n, heads, hd).transpose(1, 0, 2)
    v = (x @ Wqkv[2] + bqkv[2]).reshape(n, heads, hd).transpose(1, 0, 2)
    s = jnp.einsum('hqd,hkd->hqk', q, k) / np.sqrt(hd)
    a = jax.nn.softmax(s, axis=-1)
    o = jnp.einsum('hqk,hkd->hqd', a, v).transpose(1, 0, 2).reshape(n, c)
    return o @ Wo + bo


def _gat(x, ei, W, a_s, a_d, b, slope=0.2):
    n = x.shape[0]
    loops = jnp.arange(n, dtype=ei.dtype)
    src = jnp.concatenate([ei[0], loops])
    dst = jnp.concatenate([ei[1], loops])
    h = x @ W
    e = (h @ a_s)[src] + (h @ a_d)[dst]
    e = jnp.where(e > 0, e, slope * e)
    emax = jax.ops.segment_max(e, dst, num_segments=n)
    emax = jnp.where(jnp.isfinite(emax), emax, 0.0)
    ex = jnp.exp(e - emax[dst])
    den = jax.ops.segment_sum(ex, dst, num_segments=n)
    alpha = ex / jnp.maximum(den[dst], 1e-16)
    out = jax.ops.segment_sum(h[src] * alpha[:, None], dst, num_segments=n)
    return out + b


def _sub(x, ei, W, b):
    n = x.shape[0]
    src, dst = ei[0], ei[1]
    s = jax.ops.segment_sum(x[src], dst, num_segments=n)
    cnt = jax.ops.segment_sum(jnp.ones((src.shape[0],), x.dtype), dst, num_segments=n)
    mean = s / jnp.maximum(cnt, 1.0)[:, None]
    return jax.nn.relu(mean @ W + b)


def _cls_body(z_ref, w0, w1, w2, w3, w4, w5, b0, b1, b2, b3, b4, b5, out_ref):
    z = z_ref[...]
    z = z @ w0[...] + b0[...]
    z = z @ w1[...] + b1[...]
    z = z @ w2[...] + b2[...]
    z = z @ w3[...] + b3[...]
    z = z @ w4[...] + b4[...]
    z = z @ w5[...] + b5[...]
    out_ref[...] = z


def _classifier(z, ws, bs):
    return pl.pallas_call(
        _cls_body,
        out_shape=jax.ShapeDtypeStruct((B, 7), jnp.float32),
    )(z, *ws, *[b.reshape(1, -1) for b in bs])


def kernel(x, se, seven_edge_index, edge_index, train_edge_id, gps_gcn_w, gps_gcn_b, attn_qkv_w, attn_qkv_b, attn_out_w, attn_out_b, mlp_w1, mlp_b1, mlp_w2, mlp_b2, gps_bn_g, gps_bn_b, lin_w, lin_b, gat_w, gat_b, gat_att_src, gat_att_dst, sub_w, sub_b, gcn2_w, gcn2_b, blk_bn_g, blk_bn_b, cls_w0, cls_b0, cls_w1, cls_b1, cls_w2, cls_b2, cls_w3, cls_b3, cls_w4, cls_b4, cls_w5, cls_b5):
    x0 = x
    outs = [x0]
    for i in range(C):
        ei = seven_edge_index[i]
        t = jnp.concatenate([x0, se[i]], axis=1)
        h1 = _gcn(t, ei, gps_gcn_w[i], gps_gcn_b[i])
        h1 = _bn(h1 + t, gps_bn_g[i, 0], gps_bn_b[i, 0])
        h2 = _mha(t, attn_qkv_w[i], attn_qkv_b[i], attn_out_w[i], attn_out_b[i], HEADS)
        h2 = _bn(h2 + t, gps_bn_g[i, 1], gps_bn_b[i, 1])
        h = h1 + h2
        h = h + (jax.nn.relu(h @ mlp_w1[i] + mlp_b1[i]) @ mlp_w2[i] + mlp_b2[i])
        h = _bn(h, gps_bn_g[i, 2], gps_bn_b[i, 2])
        t = h @ lin_w[i] + lin_b[i]
        t = _gat(t, ei, gat_w[i], gat_att_src[i], gat_att_dst[i], gat_b[i])
        t_sub = _sub(t, ei, sub_w[i], sub_b[i])
        t = _gcn(t, ei, gcn2_w[i], gcn2_b[i])
        t = _bn(t + t_sub, blk_bn_g[i], blk_bn_b[i])
        outs.append(t)
    xcat = jnp.concatenate(outs, axis=1)
    node_id = edge_index[:, train_edge_id]
    z = xcat[node_id[0]] * xcat[node_id[1]]
    return _classifier(z, (cls_w0, cls_w1, cls_w2, cls_w3, cls_w4, cls_w5),
                       (cls_b0, cls_b1, cls_b2, cls_b3, cls_b4, cls_b5))


# speedup vs baseline: 9.5663x; 9.5572x over previous
import functools

import jax
import jax.numpy as jnp
import numpy as np
from jax import lax
from jax.experimental import pallas as pl
from jax.experimental.pallas import tpu as pltpu
from jax.experimental.pallas import tpu_sc as plsc

N = 2048; E = 65536; C = 7; H = 128; SE = 20; D = H + SE; B = 4096; HEADS = 4
HD = 37; HDP = 40; DP = 160
NC = 2; NS = 16; L = 16
NW = NC * NS
EH = E // NC
ECH = 4096
NCH = EH // ECH
NG = ECH // L
EW = E // NW

_mesh = plsc.VectorSubcoreMesh(core_axis_name="c", subcore_axis_name="s")


def _zero_rows(ref, nrows):
    z = jnp.zeros((L,), jnp.float32)
    for r in range(nrows):
        lax.fori_loop(0, N // L, lambda g, _, r=r: (ref.__setitem__((r, pl.ds(g * L, L)), z), 0)[1], 0)


def _zero_1d(ref):
    z = jnp.zeros((L,), jnp.float32)
    lax.fori_loop(0, N // L, lambda g, _: (ref.__setitem__(pl.ds(g * L, L), z), 0)[1], 0)


@functools.partial(
    pl.kernel,
    out_type=jax.ShapeDtypeStruct((C, NW, N), jnp.float32),
    mesh=_mesh,
    compiler_params=pltpu.CompilerParams(needs_layout_passes=False),
    scratch_types=[pltpu.VMEM((N,), jnp.float32), pltpu.VMEM((EW,), jnp.int32)],
)
def _k_deg(ei_hbm, out_hbm, acc_v, idx_v):
    c = lax.axis_index("c"); s = lax.axis_index("s")
    w = c * NS + s
    ones = jnp.ones((L,), jnp.float32)

    def rbody(i, _):
        _zero_1d(acc_v)
        pltpu.sync_copy(ei_hbm.at[pl.ds((2 * i + 1) * E + w * EW, EW)], idx_v)

        def body(g, _):
            d16 = idx_v[pl.ds(g * L, L)]
            plsc.addupdate_scatter(acc_v, [d16], ones)
            return 0
        lax.fori_loop(0, EW // L, body, 0)
        pltpu.sync_copy(acc_v, out_hbm.at[i, w])
        return 0
    lax.fori_loop(0, C, rbody, 0)


@functools.partial(
    pl.kernel,
    out_type=jax.ShapeDtypeStruct((NC, C, NS, 10, N), jnp.float32),
    mesh=_mesh,
    compiler_params=pltpu.CompilerParams(needs_layout_passes=False),
    scratch_types=[pltpu.VMEM((10, N), jnp.float32), pltpu.VMEM((10, N), jnp.float32),
                   pltpu.VMEM((ECH,), jnp.int32), pltpu.VMEM((ECH,), jnp.int32)],
)
def _k_gcn(tbl_hbm, ei_hbm, out_hbm, tbl_v, acc_v, src_v, dst_v):
    c = lax.axis_index("c"); s = lax.axis_index("s")
    rows = [jnp.full((L,), r, jnp.int32) for r in range(10)]

    def rbody(i, _):
        pltpu.sync_copy(tbl_hbm.at[i, s], tbl_v)
        _zero_rows(acc_v, 10)

        def gbody(g, _):
            s16 = src_v[pl.ds(g * L, L)]
            d16 = dst_v[pl.ds(g * L, L)]
            for r in range(10):
                v = plsc.load_gather(tbl_v, [rows[r], s16])
                plsc.addupdate_scatter(acc_v, [rows[r], d16], v)
            return 0

        def cbody(ch, _):
            off = c * EH + ch * ECH
            pltpu.sync_copy(ei_hbm.at[pl.ds(2 * i * E + off, ECH)], src_v)
            pltpu.sync_copy(ei_hbm.at[pl.ds((2 * i + 1) * E + off, ECH)], dst_v)
            lax.fori_loop(0, NG, gbody, 0)
            return 0
        lax.fori_loop(0, NCH, cbody, 0)
        pltpu.sync_copy(acc_v, out_hbm.at[c, i, s])
        return 0
    lax.fori_loop(0, C, rbody, 0)


@functools.partial(
    pl.kernel,
    out_type=(jax.ShapeDtypeStruct((NC, C, NS, 8, N), jnp.float32),
              jax.ShapeDtypeStruct((NC * C * N,), jnp.float32)),
    mesh=_mesh,
    compiler_params=pltpu.CompilerParams(needs_layout_passes=False),
    scratch_types=[pltpu.VMEM((8, N), jnp.float32), pltpu.VMEM((8, N), jnp.float32),
                   pltpu.VMEM((N,), jnp.float32), pltpu.VMEM((N,), jnp.float32),
                   pltpu.VMEM((N,), jnp.float32), pltpu.VMEM((N,), jnp.float32),
                   pltpu.VMEM((N,), jnp.float32),
                   pltpu.VMEM((ECH,), jnp.int32), pltpu.VMEM((ECH,), jnp.int32)],
)
def _k_gat(tbl_hbm, e1s_hbm, e1d_hbm, e2s_hbm, e2d_hbm, ei_hbm, out_hbm, den_hbm,
           tbl_v, acc_v, den_v, e1s_v, e1d_v, e2s_v, e2d_v, src_v, dst_v):
    c = lax.axis_index("c"); s = lax.axis_index("s")
    rows = [jnp.full((L,), r, jnp.int32) for r in range(8)]

    def rbody(i, _):
        pltpu.sync_copy(tbl_hbm.at[i, s], tbl_v)
        pltpu.sync_copy(e1s_hbm.at[pl.ds(i * N, N)], e1s_v)
        pltpu.sync_copy(e1d_hbm.at[pl.ds(i * N, N)], e1d_v)
        pltpu.sync_copy(e2s_hbm.at[pl.ds(i * N, N)], e2s_v)
        pltpu.sync_copy(e2d_hbm.at[pl.ds(i * N, N)], e2d_v)
        _zero_rows(acc_v, 8)
        _zero_1d(den_v)

        def gbody(g, _):
            s16 = src_v[pl.ds(g * L, L)]
            d16 = dst_v[pl.ds(g * L, L)]
            ex = jnp.maximum(
                plsc.load_gather(e1s_v, [s16]) * plsc.load_gather(e1d_v, [d16]),
                plsc.load_gather(e2s_v, [s16]) * plsc.load_gather(e2d_v, [d16]))
            for r in range(8):
                v = plsc.load_gather(tbl_v, [rows[r], s16]) * ex
                plsc.addupdate_scatter(acc_v, [rows[r], d16], v)

            @pl.when(s == 0)
            def _():
                plsc.addupdate_scatter(den_v, [d16], ex)
            return 0

        def cbody(ch, _):
            off = c * EH + ch * ECH
            pltpu.sync_copy(ei_hbm.at[pl.ds(2 * i * E + off, ECH)], src_v)
            pltpu.sync_copy(ei_hbm.at[pl.ds((2 * i + 1) * E + off, ECH)], dst_v)
            lax.fori_loop(0, NG, gbody, 0)
            return 0
        lax.fori_loop(0, NCH, cbody, 0)
        pltpu.sync_copy(acc_v, out_hbm.at[c, i, s])

        @pl.when(s == 0)
        def _():
            pltpu.sync_copy(den_v, den_hbm.at[pl.ds((c * C + i) * N, N)])
        return 0
    lax.fori_loop(0, C, rbody, 0)


@functools.partial(
    pl.kernel,
    out_type=(jax.ShapeDtypeStruct((NC, C, NS, 16, N), jnp.float32),
              jax.ShapeDtypeStruct((NC * C * N,), jnp.float32)),
    mesh=_mesh,
    compiler_params=pltpu.CompilerParams(needs_layout_passes=False),
    scratch_types=[pltpu.VMEM((16, N), jnp.float32), pltpu.VMEM((16, N), jnp.float32),
                   pltpu.VMEM((N,), jnp.float32),
                   pltpu.VMEM((ECH,), jnp.int32), pltpu.VMEM((ECH,), jnp.int32)],
)
def _k_sub(tbl_hbm, ei_hbm, out_hbm, cnt_hbm, tbl_v, acc_v, cnt_v, src_v, dst_v):
    c = lax.axis_index("c"); s = lax.axis_index("s")
    rows = [jnp.full((L,), r, jnp.int32) for r in range(16)]
    ones = jnp.ones((L,), jnp.float32)

    def rbody(i, _):
        pltpu.sync_copy(tbl_hbm.at[i, s], tbl_v)
        _zero_rows(acc_v, 16)
        _zero_1d(cnt_v)

        def gbody(g, _):
            s16 = src_v[pl.ds(g * L, L)]
            d16 = dst_v[pl.ds(g * L, L)]
            for r in range(16):
                v = plsc.load_gather(tbl_v, [rows[r], s16])
                plsc.addupdate_scatter(acc_v, [rows[r], d16], v)

            @pl.when(s == 0)
            def _():
                plsc.addupdate_scatter(cnt_v, [d16], ones)
            return 0

        def cbody(ch, _):
            off = c * EH + ch * ECH
            pltpu.sync_copy(ei_hbm.at[pl.ds(2 * i * E + off, ECH)], src_v)
            pltpu.sync_copy(ei_hbm.at[pl.ds((2 * i + 1) * E + off, ECH)], dst_v)
            lax.fori_loop(0, NG, gbody, 0)
            return 0
        lax.fori_loop(0, NCH, cbody, 0)
        pltpu.sync_copy(acc_v, out_hbm.at[c, i, s])

        @pl.when(s == 0)
        def _():
            pltpu.sync_copy(cnt_v, cnt_hbm.at[pl.ds((c * C + i) * N, N)])
        return 0
    lax.fori_loop(0, C, rbody, 0)


RPT = B // NW
RCH = 32

@functools.partial(
    pl.kernel,
    out_type=jax.ShapeDtypeStruct((B, 8 * H), jnp.float32),
    mesh=_mesh,
    compiler_params=pltpu.CompilerParams(needs_layout_passes=False),
    scratch_types=[pltpu.VMEM((RPT,), jnp.int32), pltpu.VMEM((RPT,), jnp.int32),
                   pltpu.VMEM((RPT,), jnp.int32),
                   pltpu.VMEM((RCH, 8 * H), jnp.float32),
                   pltpu.VMEM((RCH, 8 * H), jnp.float32),
                   pltpu.SemaphoreType.DMA],
)
def _k_pair(xcat_hbm, ei0_hbm, ei1_hbm, tid_hbm, z_hbm,
            tid_v, n0_v, n1_v, ra_v, rb_v, sem):
    c = lax.axis_index("c"); s = lax.axis_index("s")
    w = c * NS + s
    base = w * RPT
    pltpu.sync_copy(tid_hbm.at[pl.ds(base, RPT)], tid_v)
    pltpu.async_copy(ei0_hbm.at[tid_v], n0_v, sem).wait()
    pltpu.async_copy(ei1_hbm.at[tid_v], n1_v, sem).wait()
    for j in range(RPT // RCH):
        pltpu.async_copy(xcat_hbm.at[n0_v.at[pl.ds(j * RCH, RCH)]], ra_v, sem).wait()
        pltpu.async_copy(xcat_hbm.at[n1_v.at[pl.ds(j * RCH, RCH)]], rb_v, sem).wait()
        for r in range(RCH):
            def mbody(g, _, r=r):
                sl = (r, pl.ds(g * L, L))
                ra_v[sl] = ra_v[sl] * rb_v[sl]
                return 0
            lax.fori_loop(0, (8 * H) // L, mbody, 0)
        pltpu.sync_copy(ra_v, z_hbm.at[pl.ds(base + j * RCH, RCH)])


def _cls_body(z_ref, w0, w1, w2, w3, w4, w5, b0, b1, b2, b3, b4, b5, out_ref):
    z = z_ref[...]
    z = z @ w0[...] + b0[...]
    z = z @ w1[...] + b1[...]
    z = z @ w2[...] + b2[...]
    z = z @ w3[...] + b3[...]
    z = z @ w4[...] + b4[...]
    z = z @ w5[...] + b5[...]
    out_ref[...] = z


def _classifier(z, ws, bs):
    return pl.pallas_call(
        _cls_body,
        out_shape=jax.ShapeDtypeStruct((B, 7), jnp.float32),
    )(z, *ws, *[b.reshape(1, -1) for b in bs])


def _bn(xx, g, b, eps=1e-5):
    m = jnp.mean(xx, axis=0)
    v = jnp.var(xx, axis=0)
    return g * (xx - m) / jnp.sqrt(v + eps) + b


def _mha_ref(x, Wqkv, bqkv, Wo, bo, heads=HEADS):
    n, cc = x.shape
    hd = cc // heads
    q = (x @ Wqkv[0] + bqkv[0]).reshape(n, heads, hd).transpose(1, 0, 2)
    k = (x @ Wqkv[1] + bqkv[1]).reshape(n, heads, hd).transpose(1, 0, 2)
    v = (x @ Wqkv[2] + bqkv[2]).reshape(n, heads, hd).transpose(1, 0, 2)
    s = jnp.einsum('hqd,hkd->hqk', q, k) / np.sqrt(hd)
    a = jax.nn.softmax(s, axis=-1)
    o = jnp.einsum('hqk,hkd->hqd', a, v).transpose(1, 0, 2).reshape(n, cc)
    return o @ Wo + bo


def kernel(x, se, seven_edge_index, edge_index, train_edge_id, gps_gcn_w, gps_gcn_b, attn_qkv_w, attn_qkv_b, attn_out_w, attn_out_b, mlp_w1, mlp_b1, mlp_w2, mlp_b2, gps_bn_g, gps_bn_b, lin_w, lin_b, gat_w, gat_b, gat_att_src, gat_att_dst, sub_w, sub_b, gcn2_w, gcn2_b, blk_bn_g, blk_bn_b, cls_w0, cls_b0, cls_w1, cls_b1, cls_w2, cls_b2, cls_w3, cls_b3, cls_w4, cls_b4, cls_w5, cls_b5):
    ei3 = seven_edge_index
    ei = seven_edge_index.reshape(-1)

    degpart = _k_deg(ei)
    deg = degpart.sum(axis=1) + 1.0
    dinv_all = lax.rsqrt(deg)
    dinv2_all = 1.0 / deg

    g1T_list, hg_list, t_list = [], [], []
    for i in range(C):
        t = jnp.concatenate([x, se[i]], axis=1)
        hg = t @ gps_gcn_w[i]
        g1T_list.append((hg * dinv_all[i][:, None]).T)
        hg_list.append(hg)
        t_list.append(t)
    g1T = jnp.pad(jnp.stack(g1T_list), ((0, 0), (0, DP - D), (0, 0)))

    part1 = _k_gcn(g1T.reshape(C, NS, 10, N), ei)
    part1 = part1.reshape(NC, C, DP, N)
    gsum = (part1[0] + part1[1])[:, :D]

    h_list, t2_list, hgat_list = [], [], []
    e1s_l, e1d_l, e2s_l, e2d_l = [], [], [], []
    for i in range(C):
        t = t_list[i]; hg = hg_list[i]
        dinv = dinv_all[i]; dinv2 = dinv2_all[i]
        gcn1 = gsum[i].T * dinv[:, None] + hg * dinv2[:, None] + gps_gcn_b[i]
        h1 = _bn(gcn1 + t, gps_bn_g[i, 0], gps_bn_b[i, 0])
        h2 = _mha_ref(t, attn_qkv_w[i], attn_qkv_b[i], attn_out_w[i], attn_out_b[i])
        h2 = _bn(h2 + t, gps_bn_g[i, 1], gps_bn_b[i, 1])
        h = h1 + h2
        h = h + (jax.nn.relu(h @ mlp_w1[i] + mlp_b1[i]) @ mlp_w2[i] + mlp_b2[i])
        h = _bn(h, gps_bn_g[i, 2], gps_bn_b[i, 2])
        t2 = h @ lin_w[i] + lin_b[i]
        hgat = t2 @ gat_w[i]
        hs = hgat @ gat_att_src[i]
        hd = hgat @ gat_att_dst[i]
        a_sh = jnp.maximum(hs.max(), 0.0)
        b_sh = jnp.maximum(hd.max(), 0.0)
        e1s_l.append(jnp.exp(hs - a_sh)); e1d_l.append(jnp.exp(hd - b_sh))
        e2s_l.append(jnp.exp(0.2 * hs - a_sh)); e2d_l.append(jnp.exp(0.2 * hd - b_sh))
        hgat_list.append(hgat)
    e1s = jnp.stack(e1s_l); e1d = jnp.stack(e1d_l)
    e2s = jnp.stack(e2s_l); e2d = jnp.stack(e2d_l)
    hgatT = jnp.stack([hh.T for hh in hgat_list])

    msgpart, denpart = _k_gat(hgatT.reshape(C, NS, 8, N), e1s.reshape(-1),
                              e1d.reshape(-1), e2s.reshape(-1), e2d.reshape(-1), ei)
    msgpart = msgpart.reshape(NC, C, H, N)
    denpart = denpart.reshape(NC, C, N)

    t3_list, h2g_list, catT_list = [], [], []
    for i in range(C):
        hgat = hgat_list[i]
        ex_self = jnp.maximum(e1s[i] * e1d[i], e2s[i] * e2d[i])
        den = denpart[0, i] + denpart[1, i] + ex_self
        msg = (msgpart[0, i] + msgpart[1, i]).T + ex_self[:, None] * hgat
        t3 = msg / den[:, None] + gat_b[i]
        h2g = t3 @ gcn2_w[i]
        t3_list.append(t3); h2g_list.append(h2g)
        catT_list.append(jnp.concatenate([t3.T, (h2g * dinv_all[i][:, None]).T], axis=0))
    catT = jnp.stack(catT_list)

    part3, cntpart = _k_sub(catT.reshape(C, NS, 16, N), ei)
    part3 = part3.reshape(NC, C, 2 * H, N)
    s3 = part3[0] + part3[1]
    cntpart = cntpart.reshape(NC, C, N)

    outs = [x]
    for i in range(C):
        subsum = s3[i, :H].T
        g2sum = s3[i, H:].T
        cnt = cntpart[0, i] + cntpart[1, i]
        mean = subsum / jnp.maximum(cnt, 1.0)[:, None]
        t_sub = jax.nn.relu(mean @ sub_w[i] + sub_b[i])
        tt = (g2sum * dinv_all[i][:, None] + h2g_list[i] * dinv2_all[i][:, None]
              + gcn2_b[i])
        outs.append(_bn(tt + t_sub, blk_bn_g[i], blk_bn_b[i]))

    xcat = jnp.concatenate(outs, axis=1)
    z = _k_pair(xcat, edge_index[0], edge_index[1], train_edge_id)
    return _classifier(z, (cls_w0, cls_w1, cls_w2, cls_w3, cls_w4, cls_w5),
                       (cls_b0, cls_b1, cls_b2, cls_b3, cls_b4, cls_b5))


# speedup vs baseline: 18.4659x; 1.9303x over previous
import functools

import jax
import jax.numpy as jnp
import numpy as np
from jax import lax
from jax.experimental import pallas as pl
from jax.experimental.pallas import tpu as pltpu
from jax.experimental.pallas import tpu_sc as plsc

N = 2048; E = 65536; C = 7; H = 128; SE = 20; D = H + SE; B = 4096; HEADS = 4
HD = 37; HDP = 40; DP = 160
NC = 2; NS = 16; L = 16
NW = NC * NS
EH = E // NC
ECH = 4096
NCH = EH // ECH
NG = ECH // L
EW = E // NW

_mesh = plsc.VectorSubcoreMesh(core_axis_name="c", subcore_axis_name="s")


def _zero_rows(ref, nrows):
    z = jnp.zeros((L,), jnp.float32)
    for r in range(nrows):
        @plsc.parallel_loop(0, N // L, unroll=4)
        def _zb(g, r=r):
            ref[r, pl.ds(g * L, L)] = z


def _zero_1d(ref):
    z = jnp.zeros((L,), jnp.float32)

    @plsc.parallel_loop(0, N // L, unroll=4)
    def _zb(g):
        ref[pl.ds(g * L, L)] = z


@functools.partial(
    pl.kernel,
    out_type=jax.ShapeDtypeStruct((C, NW, N), jnp.float32),
    mesh=_mesh,
    compiler_params=pltpu.CompilerParams(needs_layout_passes=False),
    scratch_types=[pltpu.VMEM((N,), jnp.float32), pltpu.VMEM((EW,), jnp.int32)],
)
def _k_deg(ei_hbm, out_hbm, acc_v, idx_v):
    c = lax.axis_index("c"); s = lax.axis_index("s")
    w = c * NS + s
    ones = jnp.ones((L,), jnp.float32)

    def rbody(i, _):
        _zero_1d(acc_v)
        pltpu.sync_copy(ei_hbm.at[pl.ds((2 * i + 1) * E + w * EW, EW)], idx_v)

        @plsc.parallel_loop(0, EW // L, unroll=4)
        def body(g):
            d16 = idx_v[pl.ds(g * L, L)]
            plsc.addupdate_scatter(acc_v, [d16], ones)
        pltpu.sync_copy(acc_v, out_hbm.at[i, w])
        return 0
    lax.fori_loop(0, C, rbody, 0)


@functools.partial(
    pl.kernel,
    out_type=jax.ShapeDtypeStruct((NC, C, NS, 10, N), jnp.float32),
    mesh=_mesh,
    compiler_params=pltpu.CompilerParams(needs_layout_passes=False),
    scratch_types=[pltpu.VMEM((10, N), jnp.float32), pltpu.VMEM((10, N), jnp.float32),
                   pltpu.VMEM((ECH,), jnp.int32), pltpu.VMEM((ECH,), jnp.int32)],
)
def _k_gcn(tbl_hbm, ei_hbm, out_hbm, tbl_v, acc_v, src_v, dst_v):
    c = lax.axis_index("c"); s = lax.axis_index("s")
    rows = [jnp.full((L,), r, jnp.int32) for r in range(10)]

    def rbody(i, _):
        pltpu.sync_copy(tbl_hbm.at[i, s], tbl_v)
        _zero_rows(acc_v, 10)

        def cbody(ch, _):
            off = c * EH + ch * ECH
            pltpu.sync_copy(ei_hbm.at[pl.ds(2 * i * E + off, ECH)], src_v)
            pltpu.sync_copy(ei_hbm.at[pl.ds((2 * i + 1) * E + off, ECH)], dst_v)

            @plsc.parallel_loop(0, NG, unroll=2)
            def gbody(g):
                s16 = src_v[pl.ds(g * L, L)]
                d16 = dst_v[pl.ds(g * L, L)]
                for r in range(10):
                    v = plsc.load_gather(tbl_v, [rows[r], s16])
                    plsc.addupdate_scatter(acc_v, [rows[r], d16], v)
            return 0
        lax.fori_loop(0, NCH, cbody, 0)
        pltpu.sync_copy(acc_v, out_hbm.at[c, i, s])
        return 0
    lax.fori_loop(0, C, rbody, 0)


@functools.partial(
    pl.kernel,
    out_type=(jax.ShapeDtypeStruct((NC, C, NS, 8, N), jnp.float32),
              jax.ShapeDtypeStruct((NC * C * N,), jnp.float32)),
    mesh=_mesh,
    compiler_params=pltpu.CompilerParams(needs_layout_passes=False),
    scratch_types=[pltpu.VMEM((8, N), jnp.float32), pltpu.VMEM((8, N), jnp.float32),
                   pltpu.VMEM((N,), jnp.float32), pltpu.VMEM((N,), jnp.float32),
                   pltpu.VMEM((N,), jnp.float32), pltpu.VMEM((N,), jnp.float32),
                   pltpu.VMEM((N,), jnp.float32),
                   pltpu.VMEM((ECH,), jnp.int32), pltpu.VMEM((ECH,), jnp.int32)],
)
def _k_gat(tbl_hbm, e1s_hbm, e1d_hbm, e2s_hbm, e2d_hbm, ei_hbm, out_hbm, den_hbm,
           tbl_v, acc_v, den_v, e1s_v, e1d_v, e2s_v, e2d_v, src_v, dst_v):
    c = lax.axis_index("c"); s = lax.axis_index("s")
    rows = [jnp.full((L,), r, jnp.int32) for r in range(8)]

    def rbody(i, _):
        pltpu.sync_copy(tbl_hbm.at[i, s], tbl_v)
        pltpu.sync_copy(e1s_hbm.at[pl.ds(i * N, N)], e1s_v)
        pltpu.sync_copy(e1d_hbm.at[pl.ds(i * N, N)], e1d_v)
        pltpu.sync_copy(e2s_hbm.at[pl.ds(i * N, N)], e2s_v)
        pltpu.sync_copy(e2d_hbm.at[pl.ds(i * N, N)], e2d_v)
        _zero_rows(acc_v, 8)
        _zero_1d(den_v)

        def cbody(ch, _):
            off = c * EH + ch * ECH
            pltpu.sync_copy(ei_hbm.at[pl.ds(2 * i * E + off, ECH)], src_v)
            pltpu.sync_copy(ei_hbm.at[pl.ds((2 * i + 1) * E + off, ECH)], dst_v)

            @plsc.parallel_loop(0, NG, unroll=2)
            def gbody(g):
                s16 = src_v[pl.ds(g * L, L)]
                d16 = dst_v[pl.ds(g * L, L)]
                ex = jnp.maximum(
                    plsc.load_gather(e1s_v, [s16]) * plsc.load_gather(e1d_v, [d16]),
                    plsc.load_gather(e2s_v, [s16]) * plsc.load_gather(e2d_v, [d16]))
                for r in range(8):
                    v = plsc.load_gather(tbl_v, [rows[r], s16]) * ex
                    plsc.addupdate_scatter(acc_v, [rows[r], d16], v)

                @pl.when(s == 0)
                def _():
                    plsc.addupdate_scatter(den_v, [d16], ex)
            return 0
        lax.fori_loop(0, NCH, cbody, 0)
        pltpu.sync_copy(acc_v, out_hbm.at[c, i, s])

        @pl.when(s == 0)
        def _():
            pltpu.sync_copy(den_v, den_hbm.at[pl.ds((c * C + i) * N, N)])
        return 0
    lax.fori_loop(0, C, rbody, 0)


@functools.partial(
    pl.kernel,
    out_type=(jax.ShapeDtypeStruct((NC, C, NS, 16, N), jnp.float32),
              jax.ShapeDtypeStruct((NC * C * N,), jnp.float32)),
    mesh=_mesh,
    compiler_params=pltpu.CompilerParams(needs_layout_passes=False),
    scratch_types=[pltpu.VMEM((16, N), jnp.float32), pltpu.VMEM((16, N), jnp.float32),
                   pltpu.VMEM((N,), jnp.float32),
                   pltpu.VMEM((ECH,), jnp.int32), pltpu.VMEM((ECH,), jnp.int32)],
)
def _k_sub(tbl_hbm, ei_hbm, out_hbm, cnt_hbm, tbl_v, acc_v, cnt_v, src_v, dst_v):
    c = lax.axis_index("c"); s = lax.axis_index("s")
    rows = [jnp.full((L,), r, jnp.int32) for r in range(16)]
    ones = jnp.ones((L,), jnp.float32)

    def rbody(i, _):
        pltpu.sync_copy(tbl_hbm.at[i, s], tbl_v)
        _zero_rows(acc_v, 16)
        _zero_1d(cnt_v)

        def cbody(ch, _):
            off = c * EH + ch * ECH
            pltpu.sync_copy(ei_hbm.at[pl.ds(2 * i * E + off, ECH)], src_v)
            pltpu.sync_copy(ei_hbm.at[pl.ds((2 * i + 1) * E + off, ECH)], dst_v)

            @plsc.parallel_loop(0, NG, unroll=2)
            def gbody(g):
                s16 = src_v[pl.ds(g * L, L)]
                d16 = dst_v[pl.ds(g * L, L)]
                for r in range(16):
                    v = plsc.load_gather(tbl_v, [rows[r], s16])
                    plsc.addupdate_scatter(acc_v, [rows[r], d16], v)

                @pl.when(s == 0)
                def _():
                    plsc.addupdate_scatter(cnt_v, [d16], ones)
            return 0
        lax.fori_loop(0, NCH, cbody, 0)
        pltpu.sync_copy(acc_v, out_hbm.at[c, i, s])

        @pl.when(s == 0)
        def _():
            pltpu.sync_copy(cnt_v, cnt_hbm.at[pl.ds((c * C + i) * N, N)])
        return 0
    lax.fori_loop(0, C, rbody, 0)


RPT = B // NW
RCH = 32

@functools.partial(
    pl.kernel,
    out_type=jax.ShapeDtypeStruct((B, 8 * H), jnp.float32),
    mesh=_mesh,
    compiler_params=pltpu.CompilerParams(needs_layout_passes=False),
    scratch_types=[pltpu.VMEM((RPT,), jnp.int32), pltpu.VMEM((RPT,), jnp.int32),
                   pltpu.VMEM((RPT,), jnp.int32),
                   pltpu.VMEM((RCH, 8 * H), jnp.float32),
                   pltpu.VMEM((RCH, 8 * H), jnp.float32),
                   pltpu.SemaphoreType.DMA],
)
def _k_pair(xcat_hbm, ei0_hbm, ei1_hbm, tid_hbm, z_hbm,
            tid_v, n0_v, n1_v, ra_v, rb_v, sem):
    c = lax.axis_index("c"); s = lax.axis_index("s")
    w = c * NS + s
    base = w * RPT
    pltpu.sync_copy(tid_hbm.at[pl.ds(base, RPT)], tid_v)
    pltpu.async_copy(ei0_hbm.at[tid_v], n0_v, sem).wait()
    pltpu.async_copy(ei1_hbm.at[tid_v], n1_v, sem).wait()
    for j in range(RPT // RCH):
        pltpu.async_copy(xcat_hbm.at[n0_v.at[pl.ds(j * RCH, RCH)]], ra_v, sem).wait()
        pltpu.async_copy(xcat_hbm.at[n1_v.at[pl.ds(j * RCH, RCH)]], rb_v, sem).wait()
        for r in range(RCH):
            @plsc.parallel_loop(0, (8 * H) // L, unroll=4)
            def mbody(g, r=r):
                sl = (r, pl.ds(g * L, L))
                ra_v[sl] = ra_v[sl] * rb_v[sl]
        pltpu.sync_copy(ra_v, z_hbm.at[pl.ds(base + j * RCH, RCH)])


def _cls_body(z_ref, w0, w1, w2, w3, w4, w5, b0, b1, b2, b3, b4, b5, out_ref):
    z = z_ref[...]
    z = z @ w0[...] + b0[...]
    z = z @ w1[...] + b1[...]
    z = z @ w2[...] + b2[...]
    z = z @ w3[...] + b3[...]
    z = z @ w4[...] + b4[...]
    z = z @ w5[...] + b5[...]
    out_ref[...] = z


def _classifier(z, ws, bs):
    return pl.pallas_call(
        _cls_body,
        out_shape=jax.ShapeDtypeStruct((B, 7), jnp.float32),
    )(z, *ws, *[b.reshape(1, -1) for b in bs])


def _bn(xx, g, b, eps=1e-5):
    m = jnp.mean(xx, axis=0)
    v = jnp.var(xx, axis=0)
    return g * (xx - m) / jnp.sqrt(v + eps) + b


def _mha_ref(x, Wqkv, bqkv, Wo, bo, heads=HEADS):
    n, cc = x.shape
    hd = cc // heads
    q = (x @ Wqkv[0] + bqkv[0]).reshape(n, heads, hd).transpose(1, 0, 2)
    k = (x @ Wqkv[1] + bqkv[1]).reshape(n, heads, hd).transpose(1, 0, 2)
    v = (x @ Wqkv[2] + bqkv[2]).reshape(n, heads, hd).transpose(1, 0, 2)
    s = jnp.einsum('hqd,hkd->hqk', q, k) / np.sqrt(hd)
    a = jax.nn.softmax(s, axis=-1)
    o = jnp.einsum('hqk,hkd->hqd', a, v).transpose(1, 0, 2).reshape(n, cc)
    return o @ Wo + bo


def kernel(x, se, seven_edge_index, edge_index, train_edge_id, gps_gcn_w, gps_gcn_b, attn_qkv_w, attn_qkv_b, attn_out_w, attn_out_b, mlp_w1, mlp_b1, mlp_w2, mlp_b2, gps_bn_g, gps_bn_b, lin_w, lin_b, gat_w, gat_b, gat_att_src, gat_att_dst, sub_w, sub_b, gcn2_w, gcn2_b, blk_bn_g, blk_bn_b, cls_w0, cls_b0, cls_w1, cls_b1, cls_w2, cls_b2, cls_w3, cls_b3, cls_w4, cls_b4, cls_w5, cls_b5):
    ei3 = seven_edge_index
    ei = seven_edge_index.reshape(-1)

    degpart = _k_deg(ei)
    deg = degpart.sum(axis=1) + 1.0
    dinv_all = lax.rsqrt(deg)
    dinv2_all = 1.0 / deg

    g1T_list, hg_list, t_list = [], [], []
    for i in range(C):
        t = jnp.concatenate([x, se[i]], axis=1)
        hg = t @ gps_gcn_w[i]
        g1T_list.append((hg * dinv_all[i][:, None]).T)
        hg_list.append(hg)
        t_list.append(t)
    g1T = jnp.pad(jnp.stack(g1T_list), ((0, 0), (0, DP - D), (0, 0)))

    part1 = _k_gcn(g1T.reshape(C, NS, 10, N), ei)
    part1 = part1.reshape(NC, C, DP, N)
    gsum = (part1[0] + part1[1])[:, :D]

    h_list, t2_list, hgat_list = [], [], []
    e1s_l, e1d_l, e2s_l, e2d_l = [], [], [], []
    for i in range(C):
        t = t_list[i]; hg = hg_list[i]
        dinv = dinv_all[i]; dinv2 = dinv2_all[i]
        gcn1 = gsum[i].T * dinv[:, None] + hg * dinv2[:, None] + gps_gcn_b[i]
        h1 = _bn(gcn1 + t, gps_bn_g[i, 0], gps_bn_b[i, 0])
        h2 = _mha_ref(t, attn_qkv_w[i], attn_qkv_b[i], attn_out_w[i], attn_out_b[i])
        h2 = _bn(h2 + t, gps_bn_g[i, 1], gps_bn_b[i, 1])
        h = h1 + h2
        h = h + (jax.nn.relu(h @ mlp_w1[i] + mlp_b1[i]) @ mlp_w2[i] + mlp_b2[i])
        h = _bn(h, gps_bn_g[i, 2], gps_bn_b[i, 2])
        t2 = h @ lin_w[i] + lin_b[i]
        hgat = t2 @ gat_w[i]
        hs = hgat @ gat_att_src[i]
        hd = hgat @ gat_att_dst[i]
        a_sh = jnp.maximum(hs.max(), 0.0)
        b_sh = jnp.maximum(hd.max(), 0.0)
        e1s_l.append(jnp.exp(hs - a_sh)); e1d_l.append(jnp.exp(hd - b_sh))
        e2s_l.append(jnp.exp(0.2 * hs - a_sh)); e2d_l.append(jnp.exp(0.2 * hd - b_sh))
        hgat_list.append(hgat)
    e1s = jnp.stack(e1s_l); e1d = jnp.stack(e1d_l)
    e2s = jnp.stack(e2s_l); e2d = jnp.stack(e2d_l)
    hgatT = jnp.stack([hh.T for hh in hgat_list])

    msgpart, denpart = _k_gat(hgatT.reshape(C, NS, 8, N), e1s.reshape(-1),
                              e1d.reshape(-1), e2s.reshape(-1), e2d.reshape(-1), ei)
    msgpart = msgpart.reshape(NC, C, H, N)
    denpart = denpart.reshape(NC, C, N)

    t3_list, h2g_list, catT_list = [], [], []
    for i in range(C):
        hgat = hgat_list[i]
        ex_self = jnp.maximum(e1s[i] * e1d[i], e2s[i] * e2d[i])
        den = denpart[0, i] + denpart[1, i] + ex_self
        msg = (msgpart[0, i] + msgpart[1, i]).T + ex_self[:, None] * hgat
        t3 = msg / den[:, None] + gat_b[i]
        h2g = t3 @ gcn2_w[i]
        t3_list.append(t3); h2g_list.append(h2g)
        catT_list.append(jnp.concatenate([t3.T, (h2g * dinv_all[i][:, None]).T], axis=0))
    catT = jnp.stack(catT_list)

    part3, cntpart = _k_sub(catT.reshape(C, NS, 16, N), ei)
    part3 = part3.reshape(NC, C, 2 * H, N)
    s3 = part3[0] + part3[1]
    cntpart = cntpart.reshape(NC, C, N)

    outs = [x]
    for i in range(C):
        subsum = s3[i, :H].T
        g2sum = s3[i, H:].T
        cnt = cntpart[0, i] + cntpart[1, i]
        mean = subsum / jnp.maximum(cnt, 1.0)[:, None]
        t_sub = jax.nn.relu(mean @ sub_w[i] + sub_b[i])
        tt = (g2sum * dinv_all[i][:, None] + h2g_list[i] * dinv2_all[i][:, None]
              + gcn2_b[i])
        outs.append(_bn(tt + t_sub, blk_bn_g[i], blk_bn_b[i]))

    xcat = jnp.concatenate(outs, axis=1)
    z = _k_pair(xcat, edge_index[0], edge_index[1], train_edge_id)
    return _classifier(z, (cls_w0, cls_w1, cls_w2, cls_w3, cls_w4, cls_w5),
                       (cls_b0, cls_b1, cls_b2, cls_b3, cls_b4, cls_b5))
